# baseline (speedup 1.0000x reference)
import functools

import jax
import jax.numpy as jnp
from jax.experimental import pallas as pl
from jax.experimental.pallas import tpu as pltpu

_TOPK = 4


def _router_block(x_ref, wt_ref, b_ref, full_ref, idx_ref, sc_ref, *,
                  blk, e):
    i = pl.program_id(0)

    logits = jnp.dot(x_ref[...], wt_ref[...],
                     preferred_element_type=jnp.float32) + b_ref[...]
    m = jnp.max(logits, axis=-1, keepdims=True)
    ex = jnp.exp(logits - m)
    scores_new = ex / jnp.sum(ex, axis=-1, keepdims=True)

    scores = sc_ref[(i + 1) % 2]
    iota_f = jax.lax.broadcasted_iota(jnp.int32, (blk, e), 1).astype(jnp.float32)
    work = scores
    idx_cols = []
    for _ in range(_TOPK):
        mx = jnp.max(work, axis=-1, keepdims=True)
        idx = jnp.min(jnp.where(work == mx, iota_f, float(e)),
                      axis=-1, keepdims=True)
        work = jnp.where(iota_f == idx, -jnp.inf, work)
        idx_cols.append(idx)

    sc_ref[i % 2] = scores_new
    full_ref[...] = jnp.where(work < 0.0, scores, 0.0)
    idx_ref[...] = jnp.concatenate(idx_cols, axis=1).astype(jnp.int32)


@jax.jit
def kernel(hidden_states, weight, bias):
    b, s, h = hidden_states.shape
    e = weight.shape[0]
    n = b * s
    blk = 512
    nb = n // blk
    flat = hidden_states.reshape(n, h)
    wt = weight.T
    bias2 = bias.reshape(1, e)

    grid = (nb + 1,)
    full, idx = pl.pallas_call(
        functools.partial(_router_block, blk=blk, e=e),
        grid=grid,
        in_specs=[
            pl.BlockSpec((blk, h), lambda i: (jnp.minimum(i, nb - 1), 0)),
            pl.BlockSpec((h, e), lambda i: (0, 0)),
            pl.BlockSpec((1, e), lambda i: (0, 0)),
        ],
        out_specs=[
            pl.BlockSpec((blk, e), lambda i: (jnp.maximum(i - 1, 0), 0)),
            pl.BlockSpec((blk, _TOPK), lambda i: (jnp.maximum(i - 1, 0), 0)),
        ],
        out_shape=[
            jax.ShapeDtypeStruct((n, e), jnp.float32),
            jax.ShapeDtypeStruct((n, _TOPK), jnp.int32),
        ],
        scratch_shapes=[pltpu.VMEM((2, blk, e), jnp.float32)],
    )(flat, wt, bias2)
    return (full, idx)


# speedup vs baseline: 1.1399x; 1.1399x over previous
import functools

import jax
import jax.numpy as jnp
from jax.experimental import pallas as pl
from jax.experimental.pallas import tpu as pltpu

_TOPK = 4


def _router_block(x_ref, wt_ref, b_ref, full_ref, idx_ref, sc_ref, *,
                  blk, e):
    i = pl.program_id(0)

    logits = jnp.dot(x_ref[...], wt_ref[...],
                     preferred_element_type=jnp.float32) + b_ref[...]
    m = jnp.max(logits, axis=-1, keepdims=True)
    ex = jnp.exp(logits - m)
    scores_new = ex / jnp.sum(ex, axis=-1, keepdims=True)

    scores = sc_ref[(i + 1) % 2]
    iota_f = jax.lax.broadcasted_iota(jnp.int32, (blk, e), 1).astype(jnp.float32)
    work = scores
    idx_cols = []
    for _ in range(_TOPK):
        mx = jnp.max(work, axis=-1, keepdims=True)
        idx = jnp.min(jnp.where(work == mx, iota_f, float(e)),
                      axis=-1, keepdims=True)
        work = jnp.where(iota_f == idx, -jnp.inf, work)
        idx_cols.append(idx)

    sc_ref[i % 2] = scores_new
    full_ref[...] = jnp.where(work < 0.0, scores, 0.0)
    idx_ref[...] = jnp.concatenate(idx_cols, axis=1).astype(jnp.int32)


@jax.jit
def kernel(hidden_states, weight, bias):
    b, s, h = hidden_states.shape
    e = weight.shape[0]
    n = b * s
    blk = 1024
    nb = n // blk
    flat = hidden_states.reshape(n, h)
    wt = weight.T
    bias2 = bias.reshape(1, e)

    grid = (nb + 1,)
    full, idx = pl.pallas_call(
        functools.partial(_router_block, blk=blk, e=e),
        grid=grid,
        in_specs=[
            pl.BlockSpec((blk, h), lambda i: (jnp.minimum(i, nb - 1), 0)),
            pl.BlockSpec((h, e), lambda i: (0, 0)),
            pl.BlockSpec((1, e), lambda i: (0, 0)),
        ],
        out_specs=[
            pl.BlockSpec((blk, e), lambda i: (jnp.maximum(i - 1, 0), 0)),
            pl.BlockSpec((blk, _TOPK), lambda i: (jnp.maximum(i - 1, 0), 0)),
        ],
        out_shape=[
            jax.ShapeDtypeStruct((n, e), jnp.float32),
            jax.ShapeDtypeStruct((n, _TOPK), jnp.int32),
        ],
        scratch_shapes=[pltpu.VMEM((2, blk, e), jnp.float32)],
    )(flat, wt, bias2)
    return (full, idx)
